# baseline (speedup 1.0000x reference)
import math

import jax
import jax.numpy as jnp
from jax import lax
from jax.experimental import pallas as pl
from jax.experimental.pallas import tpu as pltpu
from jax.experimental.pallas import tpu_sc as plsc

_EMB = 64
_SCALE = math.sqrt(_EMB)
_NC, _NS = 2, 16
_NW = _NC * _NS
_BB = 128
_L = 200
_B = 4096


def _transpose_unit(rows, stage, b):
    iot = lax.iota(jnp.int32, 16)
    for a in range(_EMB // 8):
        for r in range(8):
            f = 8 * a + r
            fvec = jnp.full((16,), f, jnp.int32)
            for cg in range(_BB // 16):
                cvec = iot + (cg * 16)
                v = plsc.load_gather(rows.at[b], [cvec, fvec])
                stage[b, a, r, pl.ds(cg * 16, 16)] = v * _SCALE


def _sc_body(tok_hbm, table_hbm, out_hbm, idx_v, rows, stage, gsem0, gsem1,
             wsem0, wsem1):
    w = lax.axis_index("s") * _NC + lax.axis_index("c")
    gsems = (gsem0, gsem1)
    wsems = (wsem0, wsem1)

    pltpu.sync_copy(tok_hbm.at[:, pl.ds(w * _BB, _BB)], idx_v)

    def gather(l, b):
        pltpu.async_copy(table_hbm.at[idx_v.at[l]], rows.at[b], gsems[b])

    def wait_gather(b):
        pltpu.make_async_copy(
            table_hbm.at[idx_v.at[0]], rows.at[b], gsems[b]).wait()

    def store(l, b):
        pltpu.async_copy(stage.at[b], out_hbm.at[l, :, w], wsems[b])

    def wait_store(b):
        pltpu.make_async_copy(
            stage.at[b], out_hbm.at[0, :, w], wsems[b]).wait()

    for b in range(2):
        gather(b, b)
    for b in range(2):
        wait_gather(b)
        _transpose_unit(rows, stage, b)
        store(b, b)
        gather(b + 2, b)

    def pair(p, carry):
        for b in range(2):
            l = 2 * p + b
            wait_gather(b)
            wait_store(b)
            _transpose_unit(rows, stage, b)
            store(l, b)

            @pl.when(p < _L // 2 - 1)
            def _():
                gather(l + 2, b)
        return carry

    lax.fori_loop(1, _L // 2, pair, 0)

    for b in range(2):
        wait_store(b)


def kernel(tokens, table):
    tok_t = tokens.T.astype(jnp.int32)
    mesh = plsc.VectorSubcoreMesh(
        core_axis_name="c", subcore_axis_name="s",
        num_cores=_NC, num_subcores=_NS,
    )
    run = pl.kernel(
        _sc_body,
        out_type=jax.ShapeDtypeStruct(
            (_L, _EMB // 8, _B // _BB, 8, _BB), jnp.float32),
        mesh=mesh,
        scratch_types=[
            pltpu.VMEM((_L, _BB), jnp.int32),
            pltpu.VMEM((2, _BB, _EMB), jnp.float32),
            pltpu.VMEM((2, _EMB // 8, 8, _BB), jnp.float32),
            pltpu.SemaphoreType.DMA,
            pltpu.SemaphoreType.DMA,
            pltpu.SemaphoreType.DMA,
            pltpu.SemaphoreType.DMA,
        ],
        compiler_params=pltpu.CompilerParams(
            use_tc_tiling_on_sc=False, needs_layout_passes=False),
    )
    out5 = run(tok_t, table)
    return out5.transpose(2, 4, 0, 1, 3).reshape(_B, _L, _EMB)


# speedup vs baseline: 1.5854x; 1.5854x over previous
import math

import jax
import jax.numpy as jnp
from jax import lax
from jax.experimental import pallas as pl
from jax.experimental.pallas import tpu as pltpu
from jax.experimental.pallas import tpu_sc as plsc

_EMB = 64
_SCALE = math.sqrt(_EMB)
_NC, _NS = 2, 16
_NW = _NC * _NS
_L = 200
_B = 4096
_BPW = _B // _NW


def _sc_body(idx_hbm, table_hbm, out_hbm, idx_all, rows, gsem0, gsem1):
    w = lax.axis_index("s") * _NC + lax.axis_index("c")
    base = w * _BPW
    gsems = (gsem0, gsem1)

    pltpu.sync_copy(idx_hbm.at[pl.ds(base * _L, _BPW * _L)], idx_all)

    def gather(g, b):
        pltpu.async_copy(
            table_hbm.at[idx_all.at[pl.ds(g * _L, _L)]], rows.at[b], gsems[b])

    for b in range(2):
        gather(b, b)

    def pair(p, carry):
        for b in range(2):
            g = 2 * p + b
            pltpu.make_async_copy(
                table_hbm.at[idx_all.at[pl.ds(0, _L)]], rows.at[b],
                gsems[b]).wait()

            def scale(r, c2):
                for j in range(_EMB // 16):
                    sl = pl.ds(j * 16, 16)
                    rows[b, r, sl] = rows[b, r, sl] * _SCALE
                return c2

            lax.fori_loop(0, _L, scale, 0, unroll=4)
            pltpu.sync_copy(rows.at[b], out_hbm.at[base + g])

            @pl.when(g + 2 < _BPW)
            def _():
                gather(g + 2, b)
        return carry

    lax.fori_loop(0, _BPW // 2, pair, 0)


def kernel(tokens, table):
    idx = tokens.reshape(-1).astype(jnp.int32)
    mesh = plsc.VectorSubcoreMesh(
        core_axis_name="c", subcore_axis_name="s",
        num_cores=_NC, num_subcores=_NS,
    )
    run = pl.kernel(
        _sc_body,
        out_type=jax.ShapeDtypeStruct((_B, _L, _EMB), jnp.float32),
        mesh=mesh,
        scratch_types=[
            pltpu.VMEM((_BPW * _L,), jnp.int32),
            pltpu.VMEM((2, _L, _EMB), jnp.float32),
            pltpu.SemaphoreType.DMA,
            pltpu.SemaphoreType.DMA,
        ],
        compiler_params=pltpu.CompilerParams(use_tc_tiling_on_sc=False),
    )
    return run(idx, table)


# speedup vs baseline: 1.7927x; 1.1308x over previous
import math

import jax
import jax.numpy as jnp
from jax import lax
from jax.experimental import pallas as pl
from jax.experimental.pallas import tpu as pltpu
from jax.experimental.pallas import tpu_sc as plsc

_EMB = 64
_SCALE = math.sqrt(_EMB)
_NC, _NS = 2, 16
_NW = _NC * _NS
_BB = 128
_L = 200
_B = 4096
_SP = _BB + 1


def _transpose_unit(rows, stage, b):
    iot = lax.iota(jnp.int32, 16)

    def col(c, carry):
        for f0 in range(0, _EMB, 16):
            fvec = iot + f0
            v = rows[b, c, pl.ds(f0, 16)] * _SCALE
            plsc.store_scatter(stage.at[b], [fvec, jnp.full((16,), 0, jnp.int32) + c], v)
        return carry

    lax.fori_loop(0, _BB, col, 0, unroll=4)


def _sc_body(tok_hbm, table_hbm, out_hbm, idx_v, rows, stage, gsem0, gsem1,
             wsem0, wsem1):
    w = lax.axis_index("s") * _NC + lax.axis_index("c")
    gsems = (gsem0, gsem1)
    wsems = (wsem0, wsem1)

    pltpu.sync_copy(tok_hbm.at[:, pl.ds(w * _BB, _BB)], idx_v)

    def gather(l, b):
        pltpu.async_copy(table_hbm.at[idx_v.at[l]], rows.at[b], gsems[b])

    def wait_gather(b):
        pltpu.make_async_copy(
            table_hbm.at[idx_v.at[0]], rows.at[b], gsems[b]).wait()

    def store(l, b):
        for a in range(_EMB // 8):
            pltpu.async_copy(
                stage.at[b, pl.ds(8 * a, 8), pl.ds(0, _BB)],
                out_hbm.at[l, a, w], wsems[b])

    def wait_store(b):
        for a in range(_EMB // 8):
            pltpu.make_async_copy(
                stage.at[b, pl.ds(8 * a, 8), pl.ds(0, _BB)],
                out_hbm.at[0, a, w], wsems[b]).wait()

    for b in range(2):
        gather(b, b)
    for b in range(2):
        wait_gather(b)
        _transpose_unit(rows, stage, b)
        store(b, b)
        gather(b + 2, b)

    def pair(p, carry):
        for b in range(2):
            l = 2 * p + b
            wait_gather(b)
            wait_store(b)
            _transpose_unit(rows, stage, b)
            store(l, b)

            @pl.when(p < _L // 2 - 1)
            def _():
                gather(l + 2, b)
        return carry

    lax.fori_loop(1, _L // 2, pair, 0)

    for b in range(2):
        wait_store(b)


def kernel(tokens, table):
    tok_t = tokens.T.astype(jnp.int32)
    mesh = plsc.VectorSubcoreMesh(
        core_axis_name="c", subcore_axis_name="s",
        num_cores=_NC, num_subcores=_NS,
    )
    run = pl.kernel(
        _sc_body,
        out_type=jax.ShapeDtypeStruct(
            (_L, _EMB // 8, _B // _BB, 8, _BB), jnp.float32),
        mesh=mesh,
        scratch_types=[
            pltpu.VMEM((_L, _BB), jnp.int32),
            pltpu.VMEM((2, _BB, _EMB), jnp.float32),
            pltpu.VMEM((2, _EMB, _SP), jnp.float32),
            pltpu.SemaphoreType.DMA,
            pltpu.SemaphoreType.DMA,
            pltpu.SemaphoreType.DMA,
            pltpu.SemaphoreType.DMA,
        ],
        compiler_params=pltpu.CompilerParams(
            use_tc_tiling_on_sc=False, needs_layout_passes=False),
    )
    out5 = run(tok_t, table)
    return out5.transpose(2, 4, 0, 1, 3).reshape(_B, _L, _EMB)


# speedup vs baseline: 2.1796x; 1.2158x over previous
import math

import jax
import jax.numpy as jnp
from jax import lax
from jax.experimental import pallas as pl
from jax.experimental.pallas import tpu as pltpu
from jax.experimental.pallas import tpu_sc as plsc

_EMB = 64
_SCALE = math.sqrt(_EMB)
_NC, _NS = 2, 16
_NW = _NC * _NS
_BB = 128
_L = 200
_B = 4096
_SP = _BB + 1


def _transpose_unit(rows, stage, b):
    iot = lax.iota(jnp.int32, 16)
    avecs = [iot // 8 + (f0 // 8) for f0 in range(0, _EMB, 16)]
    rvecs = [iot % 8 for _ in range(0, _EMB, 16)]
    zero = jnp.zeros((16,), jnp.int32)

    def col(c, carry):
        cvec = zero + c
        vs = [rows[b, c, pl.ds(f0, 16)] * _SCALE
              for f0 in range(0, _EMB, 16)]
        for k in range(_EMB // 16):
            plsc.store_scatter(stage.at[b], [avecs[k], rvecs[k], cvec], vs[k])
        return carry

    lax.fori_loop(0, _BB, col, 0, unroll=4)


def _sc_body(tok_hbm, table_hbm, out_hbm, idx_v, rows, stage, gsem0, gsem1,
             wsem0, wsem1):
    w = lax.axis_index("s") * _NC + lax.axis_index("c")
    gsems = (gsem0, gsem1)
    wsems = (wsem0, wsem1)

    pltpu.sync_copy(tok_hbm.at[:, pl.ds(w * _BB, _BB)], idx_v)

    def gather(l, b):
        pltpu.async_copy(table_hbm.at[idx_v.at[l]], rows.at[b], gsems[b])

    def wait_gather(b):
        pltpu.make_async_copy(
            table_hbm.at[idx_v.at[0]], rows.at[b], gsems[b]).wait()

    def store(l, b):
        pltpu.async_copy(
            stage.at[b, :, :, pl.ds(0, _BB)], out_hbm.at[l, :, w], wsems[b])

    def wait_store(b):
        pltpu.make_async_copy(
            stage.at[b, :, :, pl.ds(0, _BB)], out_hbm.at[0, :, w],
            wsems[b]).wait()

    for b in range(2):
        gather(b, b)
    for b in range(2):
        wait_gather(b)
        _transpose_unit(rows, stage, b)
        store(b, b)
        gather(b + 2, b)

    def pair(p, carry):
        for b in range(2):
            l = 2 * p + b
            wait_gather(b)
            wait_store(b)
            _transpose_unit(rows, stage, b)
            store(l, b)

            @pl.when(p < _L // 2 - 1)
            def _():
                gather(l + 2, b)
        return carry

    lax.fori_loop(1, _L // 2, pair, 0)

    for b in range(2):
        wait_store(b)


def kernel(tokens, table):
    tok_t = tokens.T.astype(jnp.int32)
    mesh = plsc.VectorSubcoreMesh(
        core_axis_name="c", subcore_axis_name="s",
        num_cores=_NC, num_subcores=_NS,
    )
    run = pl.kernel(
        _sc_body,
        out_type=jax.ShapeDtypeStruct(
            (_L, _EMB // 8, _B // _BB, 8, _BB), jnp.float32),
        mesh=mesh,
        scratch_types=[
            pltpu.VMEM((_L, _BB), jnp.int32),
            pltpu.VMEM((2, _BB, _EMB), jnp.float32),
            pltpu.VMEM((2, _EMB // 8, 8, _SP), jnp.float32),
            pltpu.SemaphoreType.DMA,
            pltpu.SemaphoreType.DMA,
            pltpu.SemaphoreType.DMA,
            pltpu.SemaphoreType.DMA,
        ],
        compiler_params=pltpu.CompilerParams(
            use_tc_tiling_on_sc=False, needs_layout_passes=False),
    )
    out5 = run(tok_t, table)
    return out5.transpose(2, 4, 0, 1, 3).reshape(_B, _L, _EMB)
